# baseline (speedup 1.0000x reference)
import dataclasses

import jax
import jax.numpy as jnp
from jax import lax
from jax.experimental import pallas as pl
from jax.experimental.pallas import tpu as pltpu
from jax.experimental.pallas import tpu_sc as plsc

D = 128
DP = D // 2
NWIN = D // 32
L_SEQ = 50
L_PAD = 56
N_ROWS = 4096 * L_SEQ
NC, NS = 2, 16
NW = NC * NS
ROWS_PER_W = N_ROWS // NW
BLK = 100
N_BLK = ROWS_PER_W // BLK
B_PER_BLK = BLK // L_SEQ
IDX_PAD = 112
IDX_GROUPS = IDX_PAD // 16
T1_ROWS, T2_ROWS = 288, 217


def _div_const(n, c):
    q = (n.astype(jnp.float32) * jnp.float32(1.0 / c)).astype(jnp.int32)
    r = n - q * c
    q = jnp.where(r < 0, q - 1, q)
    r = jnp.where(r < 0, r + c, r)
    q = jnp.where(r >= c, q + 1, q)
    return q


def _mod_const(n, c):
    return n - _div_const(n, c) * c


def _combine_tables_kernel(hour_ref, weekday_ref, day_ref, month_ref,
                           t1_ref, t2_ref):
    m = month_ref[...]
    d = day_ref[...]
    for h in range(24):
        t1_ref[h * 12:(h + 1) * 12, :] = (
            m[:12, :] + hour_ref[h:h + 1, :]).astype(jnp.bfloat16)
    for w in range(7):
        t2_ref[w * 31:(w + 1) * 31, :] = (
            d[:31, :] + weekday_ref[w:w + 1, :]).astype(jnp.bfloat16)


def _combine_tables(hour_table, weekday_table, day_table, month_table):
    return pl.pallas_call(
        _combine_tables_kernel,
        out_shape=(
            jax.ShapeDtypeStruct((T1_ROWS, D), jnp.bfloat16),
            jax.ShapeDtypeStruct((T2_ROWS, D), jnp.bfloat16),
        ),
    )(hour_table, weekday_table, day_table, month_table)


def _pack_pairs(tb, rows):
    perm = tb.reshape(rows, NWIN, 2, 16).transpose(0, 1, 3, 2)
    packed = lax.bitcast_convert_type(perm, jnp.int32).reshape(rows, DP)
    return jnp.concatenate(
        [packed, jnp.zeros((rows, D - DP), jnp.int32)], axis=1)


def _sc_kernel(a_hbm, b_hbm, t1_hbm, t2_hbm, out_hbm,
               a_all, b_all, i10, i20, i11, i21,
               g10, g20, g11, g21, o0, o1, t1_sh, t2_sh,
               gsem0, gsem1, osem0, osem1):
    i32 = jnp.int32
    wid = lax.axis_index("s") * i32(NC) + lax.axis_index("c")

    sets = ((i10, i20, g10, g20, o0, gsem0, osem0),
            (i11, i21, g11, g21, o1, gsem1, osem1))

    tile_base = wid * i32(N_BLK * IDX_PAD)
    pltpu.sync_copy(a_hbm.at[pl.ds(tile_base, N_BLK * IDX_PAD)], a_all)
    pltpu.sync_copy(b_hbm.at[pl.ds(tile_base, N_BLK * IDX_PAD)], b_all)

    @pl.when(lax.axis_index("s") == 0)
    def _():
        pltpu.sync_copy(t1_hbm, t1_sh)
        pltpu.sync_copy(t2_hbm, t2_sh)
    plsc.subcore_barrier()

    def prep(jj, st):
        i1_v, i2_v, g1_v, g2_v = st[:4]
        gsem = st[5]
        abase = jj * i32(IDX_PAD)

        @pl.loop(jnp.int32(0), jnp.int32(IDX_GROUPS))
        def _idx(g):
            sl = pl.ds(g.astype(i32) * i32(16), 16)
            gsl = pl.ds(abase + g.astype(i32) * i32(16), 16)
            av = a_all[gsl]
            bv = b_all[gsl]
            t = av * 216 + bv
            y = av * 16777 + _div_const(t, 1000)
            d = _div_const(y, 86400)
            hour = _div_const(_mod_const(y, 1440), 60)
            month = _mod_const(_div_const(y, 2629800), 12)
            i1 = hour * 12 + month
            i2 = _mod_const(d, 7) * 31 + _mod_const(d, 31)
            i1_v[sl] = jnp.clip(i1, 0, T1_ROWS - 1)
            i2_v[sl] = jnp.clip(i2, 0, T2_ROWS - 1)

        pltpu.async_copy(t1_sh.at[i1_v], g1_v, gsem)
        pltpu.async_copy(t2_sh.at[i2_v], g2_v, gsem)

    def consume(jj, st, first):
        g1_v, g2_v, o_v, gsem, osem = st[2], st[3], st[4], st[5], st[6]
        pltpu.make_async_copy(t1_hbm.at[pl.ds(0, IDX_PAD)], g1_v, gsem).wait()
        pltpu.make_async_copy(t2_hbm.at[pl.ds(0, IDX_PAD)], g2_v, gsem).wait()

        @pl.when(jnp.logical_not(first))
        def _():
            for i in range(B_PER_BLK):
                pltpu.make_async_copy(
                    o_v.at[pl.ds(i * L_SEQ, L_PAD)],
                    out_hbm.at[jnp.int32(0)], osem).wait()

        @pl.loop(jnp.int32(0), jnp.int32(BLK), step=jnp.int32(4))
        def _row(r):
            for u in range(4):
                ri = r.astype(i32) + i32(u)
                for w in range(NWIN):
                    v1 = g1_v[ri, pl.ds(w * 16, 16)]
                    v2 = g2_v[ri, pl.ds(w * 16, 16)]
                    s = (plsc.bitcast(v1, jnp.bfloat16)
                         + plsc.bitcast(v2, jnp.bfloat16))
                    lo, hi = plsc.unpack(s,
                                         format=plsc.PackFormat.INTERLEAVED,
                                         preferred_element_type=jnp.float32)
                    o_v[ri, pl.ds(w * 32, 16)] = lo
                    o_v[ri, pl.ds(w * 32 + 16, 16)] = hi

        base_batch = (wid * i32(ROWS_PER_W // L_SEQ)
                      + jj * i32(B_PER_BLK))
        for i in range(B_PER_BLK):
            pltpu.async_copy(o_v.at[pl.ds(i * L_SEQ, L_PAD)],
                             out_hbm.at[base_batch + i32(i)], osem)

    prep(i32(0), sets[0])

    @pl.loop(jnp.int32(0), jnp.int32(N_BLK), step=jnp.int32(2))
    def _blk(j):
        prep(j + i32(1), sets[1])
        consume(j, sets[0], j == 0)

        @pl.when(j + i32(2) < i32(N_BLK))
        def _():
            prep(j + i32(2), sets[0])
        consume(j + i32(1), sets[1], j == 0)

    for st in sets:
        o_v, osem = st[4], st[6]
        for i in range(B_PER_BLK):
            pltpu.make_async_copy(o_v.at[pl.ds(i * L_SEQ, L_PAD)],
                                  out_hbm.at[jnp.int32(0)], osem).wait()


def kernel(x, hour_table, weekday_table, day_table, month_table):
    xf = x.reshape(-1)
    a = (xf >> 24).astype(jnp.int32)
    b = (xf & 0xFFFFFF).astype(jnp.int32)
    pad = ((0, 0), (0, IDX_PAD - BLK))
    a = jnp.pad(a.reshape(NW * N_BLK, BLK), pad).reshape(-1)
    b = jnp.pad(b.reshape(NW * N_BLK, BLK), pad).reshape(-1)

    t1b, t2b = _combine_tables(hour_table, weekday_table, day_table,
                               month_table)
    t1p = _pack_pairs(t1b, T1_ROWS)
    t2p = _pack_pairs(t2b, T2_ROWS)

    cp = pltpu.CompilerParams()
    if "needs_layout_passes" in pltpu.CompilerParams.__dataclass_fields__:
        cp = dataclasses.replace(cp, needs_layout_passes=False)
    mesh = plsc.VectorSubcoreMesh(core_axis_name="c", subcore_axis_name="s")
    idx_t = pltpu.VMEM((IDX_PAD,), jnp.int32)
    g_t = pltpu.VMEM((IDX_PAD, D), jnp.int32)
    o_t = pltpu.VMEM((IDX_PAD, D), jnp.float32)
    sc = pl.kernel(
        _sc_kernel,
        mesh=mesh,
        compiler_params=cp,
        out_type=jax.ShapeDtypeStruct((x.shape[0], L_PAD, D), jnp.float32),
        scratch_types=[
            pltpu.VMEM((N_BLK * IDX_PAD,), jnp.int32),
            pltpu.VMEM((N_BLK * IDX_PAD,), jnp.int32),
            idx_t, idx_t, idx_t, idx_t,
            g_t, g_t, g_t, g_t, o_t, o_t,
            pltpu.VMEM_SHARED((T1_ROWS, D), jnp.int32),
            pltpu.VMEM_SHARED((T2_ROWS, D), jnp.int32),
            pltpu.SemaphoreType.DMA, pltpu.SemaphoreType.DMA,
            pltpu.SemaphoreType.DMA, pltpu.SemaphoreType.DMA,
        ],
    )
    return sc(a, b, t1p, t2p)[:, :L_SEQ, :]


# speedup vs baseline: 1.1349x; 1.1349x over previous
import dataclasses

import jax
import jax.numpy as jnp
from jax import lax
from jax.experimental import pallas as pl
from jax.experimental.pallas import tpu as pltpu
from jax.experimental.pallas import tpu_sc as plsc

D = 128
DP = D // 2
NWIN = D // 32
L_SEQ = 50
N_ROWS = 4096 * L_SEQ
NC, NS = 2, 16
NW = NC * NS
ROWS_PER_W = N_ROWS // NW
BLK = 100
N_BLK = ROWS_PER_W // BLK
B_PER_BLK = BLK // L_SEQ
IDX_PAD = 112
IDX_GROUPS = IDX_PAD // 16
T1_ROWS, T2_ROWS = 288, 217


def _div_const(n, c):
    q = (n.astype(jnp.float32) * jnp.float32(1.0 / c)).astype(jnp.int32)
    r = n - q * c
    q = jnp.where(r < 0, q - 1, q)
    r = jnp.where(r < 0, r + c, r)
    q = jnp.where(r >= c, q + 1, q)
    return q


def _mod_const(n, c):
    return n - _div_const(n, c) * c


def _combine_tables_kernel(hour_ref, weekday_ref, day_ref, month_ref,
                           t1_ref, t2_ref):
    m = month_ref[...]
    d = day_ref[...]
    for h in range(24):
        t1_ref[h * 12:(h + 1) * 12, :] = (
            m[:12, :] + hour_ref[h:h + 1, :]).astype(jnp.bfloat16)
    for w in range(7):
        t2_ref[w * 31:(w + 1) * 31, :] = (
            d[:31, :] + weekday_ref[w:w + 1, :]).astype(jnp.bfloat16)


def _combine_tables(hour_table, weekday_table, day_table, month_table):
    return pl.pallas_call(
        _combine_tables_kernel,
        out_shape=(
            jax.ShapeDtypeStruct((T1_ROWS, D), jnp.bfloat16),
            jax.ShapeDtypeStruct((T2_ROWS, D), jnp.bfloat16),
        ),
    )(hour_table, weekday_table, day_table, month_table)


def _pack_pairs(tb, rows):
    perm = tb.reshape(rows, NWIN, 2, 16).transpose(0, 1, 3, 2)
    packed = lax.bitcast_convert_type(perm, jnp.int32).reshape(rows, DP)
    return jnp.concatenate(
        [packed, jnp.zeros((rows, D - DP), jnp.int32)], axis=1)


def _sc_kernel(a_hbm, b_hbm, t1_hbm, t2_hbm, out_hbm,
               a_all, b_all, i10, i20, i11, i21,
               g10, g20, g11, g21, o0, o1, t1_sh, t2_sh,
               gsem0, gsem1, osem0, osem1):
    i32 = jnp.int32
    wid = lax.axis_index("s") * i32(NC) + lax.axis_index("c")

    sets = ((i10, i20, g10, g20, o0, gsem0, osem0),
            (i11, i21, g11, g21, o1, gsem1, osem1))

    tile_base = wid * i32(N_BLK * IDX_PAD)
    pltpu.sync_copy(a_hbm.at[pl.ds(tile_base, N_BLK * IDX_PAD)], a_all)
    pltpu.sync_copy(b_hbm.at[pl.ds(tile_base, N_BLK * IDX_PAD)], b_all)

    @pl.when(lax.axis_index("s") == 0)
    def _():
        pltpu.sync_copy(t1_hbm, t1_sh)
        pltpu.sync_copy(t2_hbm, t2_sh)
    plsc.subcore_barrier()

    def prep(jj, st):
        i1_v, i2_v, g1_v, g2_v = st[:4]
        gsem = st[5]
        abase = jj * i32(IDX_PAD)

        @pl.loop(jnp.int32(0), jnp.int32(IDX_GROUPS))
        def _idx(g):
            sl = pl.ds(g.astype(i32) * i32(16), 16)
            gsl = pl.ds(abase + g.astype(i32) * i32(16), 16)
            av = a_all[gsl]
            bv = b_all[gsl]
            t = av * 216 + bv
            y = av * 16777 + _div_const(t, 1000)
            d = _div_const(y, 86400)
            hour = _div_const(_mod_const(y, 1440), 60)
            month = _mod_const(_div_const(y, 2629800), 12)
            i1 = hour * 12 + month
            i2 = _mod_const(d, 7) * 31 + _mod_const(d, 31)
            i1_v[sl] = jnp.clip(i1, 0, T1_ROWS - 1)
            i2_v[sl] = jnp.clip(i2, 0, T2_ROWS - 1)

        pltpu.async_copy(t1_sh.at[i1_v], g1_v, gsem)
        pltpu.async_copy(t2_sh.at[i2_v], g2_v, gsem)

    def consume(jj, st, first):
        g1_v, g2_v, o_v, gsem, osem = st[2], st[3], st[4], st[5], st[6]
        pltpu.make_async_copy(t1_hbm.at[pl.ds(0, IDX_PAD)], g1_v, gsem).wait()
        pltpu.make_async_copy(t2_hbm.at[pl.ds(0, IDX_PAD)], g2_v, gsem).wait()

        @pl.when(jnp.logical_not(first))
        def _():
            for i in range(B_PER_BLK):
                pltpu.make_async_copy(
                    o_v.at[pl.ds(i * L_SEQ, L_SEQ)],
                    out_hbm.at[jnp.int32(0)], osem).wait()

        @pl.loop(jnp.int32(0), jnp.int32(BLK), step=jnp.int32(4))
        def _row(r):
            for u in range(4):
                ri = r.astype(i32) + i32(u)
                for w in range(NWIN):
                    v1 = g1_v[ri, pl.ds(w * 16, 16)]
                    v2 = g2_v[ri, pl.ds(w * 16, 16)]
                    s = (plsc.bitcast(v1, jnp.bfloat16)
                         + plsc.bitcast(v2, jnp.bfloat16))
                    lo, hi = plsc.unpack(s,
                                         format=plsc.PackFormat.INTERLEAVED,
                                         preferred_element_type=jnp.float32)
                    o_v[ri, pl.ds(w * 32, 16)] = lo
                    o_v[ri, pl.ds(w * 32 + 16, 16)] = hi

        base_batch = (wid * i32(ROWS_PER_W // L_SEQ)
                      + jj * i32(B_PER_BLK))
        for i in range(B_PER_BLK):
            pltpu.async_copy(o_v.at[pl.ds(i * L_SEQ, L_SEQ)],
                             out_hbm.at[base_batch + i32(i)], osem)

    prep(i32(0), sets[0])

    @pl.loop(jnp.int32(0), jnp.int32(N_BLK), step=jnp.int32(2))
    def _blk(j):
        prep(j + i32(1), sets[1])
        consume(j, sets[0], j == 0)

        @pl.when(j + i32(2) < i32(N_BLK))
        def _():
            prep(j + i32(2), sets[0])
        consume(j + i32(1), sets[1], j == 0)

    for st in sets:
        o_v, osem = st[4], st[6]
        for i in range(B_PER_BLK):
            pltpu.make_async_copy(o_v.at[pl.ds(i * L_SEQ, L_SEQ)],
                                  out_hbm.at[jnp.int32(0)], osem).wait()


def kernel(x, hour_table, weekday_table, day_table, month_table):
    xf = x.reshape(-1)
    a = (xf >> 24).astype(jnp.int32)
    b = (xf & 0xFFFFFF).astype(jnp.int32)
    pad = ((0, 0), (0, IDX_PAD - BLK))
    a = jnp.pad(a.reshape(NW * N_BLK, BLK), pad).reshape(-1)
    b = jnp.pad(b.reshape(NW * N_BLK, BLK), pad).reshape(-1)

    t1b, t2b = _combine_tables(hour_table, weekday_table, day_table,
                               month_table)
    t1p = _pack_pairs(t1b, T1_ROWS)
    t2p = _pack_pairs(t2b, T2_ROWS)

    cp = pltpu.CompilerParams()
    if "needs_layout_passes" in pltpu.CompilerParams.__dataclass_fields__:
        cp = dataclasses.replace(cp, needs_layout_passes=False)
    mesh = plsc.VectorSubcoreMesh(core_axis_name="c", subcore_axis_name="s")
    idx_t = pltpu.VMEM((IDX_PAD,), jnp.int32)
    g_t = pltpu.VMEM((IDX_PAD, D), jnp.int32)
    o_t = pltpu.VMEM((IDX_PAD, D), jnp.float32)
    sc = pl.kernel(
        _sc_kernel,
        mesh=mesh,
        compiler_params=cp,
        out_type=jax.ShapeDtypeStruct((x.shape[0], L_SEQ, D), jnp.float32),
        scratch_types=[
            pltpu.VMEM((N_BLK * IDX_PAD,), jnp.int32),
            pltpu.VMEM((N_BLK * IDX_PAD,), jnp.int32),
            idx_t, idx_t, idx_t, idx_t,
            g_t, g_t, g_t, g_t, o_t, o_t,
            pltpu.VMEM_SHARED((T1_ROWS, D), jnp.int32),
            pltpu.VMEM_SHARED((T2_ROWS, D), jnp.int32),
            pltpu.SemaphoreType.DMA, pltpu.SemaphoreType.DMA,
            pltpu.SemaphoreType.DMA, pltpu.SemaphoreType.DMA,
        ],
    )
    return sc(a, b, t1p, t2p)
